# Initial kernel scaffold; baseline (speedup 1.0000x reference)
import jax
import jax.numpy as jnp
from jax.experimental import pallas as pl


def kernel(x, pos_embed):
    raise NotImplementedError("write your pallas kernel here")


# speedup vs baseline: 2.8244x; 2.8244x over previous
import jax
import jax.numpy as jnp
from jax.experimental import pallas as pl
from jax.experimental.pallas import tpu as pltpu

_BT = 512


def _add_body(x_ref, pe_ref, o_ref):
    o_ref[...] = x_ref[...] + pe_ref[...]


def kernel(x, pos_embed):
    B, T, C = x.shape
    pe = pos_embed[:T]
    grid = (T // _BT, B)
    return pl.pallas_call(
        _add_body,
        grid=grid,
        in_specs=[
            pl.BlockSpec((1, _BT, C), lambda t, b: (b, t, 0)),
            pl.BlockSpec((_BT, C), lambda t, b: (t, 0)),
        ],
        out_specs=pl.BlockSpec((1, _BT, C), lambda t, b: (b, t, 0)),
        out_shape=jax.ShapeDtypeStruct((B, T, C), x.dtype),
        compiler_params=pltpu.CompilerParams(
            dimension_semantics=("arbitrary", "arbitrary"),
        ),
    )(x, pe)


# speedup vs baseline: 3.1329x; 1.1092x over previous
import jax
import jax.numpy as jnp
from jax.experimental import pallas as pl
from jax.experimental.pallas import tpu as pltpu

_BT = 1024


def _add_body(x_ref, pe_ref, o_ref):
    o_ref[...] = x_ref[...] + pe_ref[...]


def kernel(x, pos_embed):
    B, T, C = x.shape
    pe = pos_embed[:T]
    grid = (T // _BT, B)
    return pl.pallas_call(
        _add_body,
        grid=grid,
        in_specs=[
            pl.BlockSpec((1, _BT, C), lambda t, b: (b, t, 0)),
            pl.BlockSpec((_BT, C), lambda t, b: (t, 0)),
        ],
        out_specs=pl.BlockSpec((1, _BT, C), lambda t, b: (b, t, 0)),
        out_shape=jax.ShapeDtypeStruct((B, T, C), x.dtype),
        compiler_params=pltpu.CompilerParams(
            dimension_semantics=("arbitrary", "arbitrary"),
        ),
    )(x, pe)


# speedup vs baseline: 3.3116x; 1.0571x over previous
import jax
import jax.numpy as jnp
from jax.experimental import pallas as pl
from jax.experimental.pallas import tpu as pltpu

_BT = 2048


def _add_body(x_ref, pe_ref, o_ref):
    o_ref[...] = x_ref[...] + pe_ref[...]


def kernel(x, pos_embed):
    B, T, C = x.shape
    pe = pos_embed[:T]
    grid = (T // _BT, B)
    return pl.pallas_call(
        _add_body,
        grid=grid,
        in_specs=[
            pl.BlockSpec((1, _BT, C), lambda t, b: (b, t, 0)),
            pl.BlockSpec((_BT, C), lambda t, b: (t, 0)),
        ],
        out_specs=pl.BlockSpec((1, _BT, C), lambda t, b: (b, t, 0)),
        out_shape=jax.ShapeDtypeStruct((B, T, C), x.dtype),
        compiler_params=pltpu.CompilerParams(
            dimension_semantics=("arbitrary", "arbitrary"),
        ),
    )(x, pe)
